# baseline (speedup 1.0000x reference)
import jax
import jax.numpy as jnp
from jax import lax
from jax.experimental import pallas as pl
from jax.experimental.pallas import tpu as pltpu
from jax.experimental.pallas import tpu_sc as plsc

BATCH = 16384
NC, NS, L = 2, 16, 16
NW = NC * NS
BPW = BATCH // NW
D_CONST = 1.702


def _abc_body(item_hbm, a_hbm, b_hbm, c_hbm, a_out, b_out, c_out,
              iidx, av, bv, cv, sem):
    wid = lax.axis_index("s") * NC + lax.axis_index("c")
    base = wid * BPW
    pltpu.sync_copy(item_hbm.at[pl.ds(base, BPW)], iidx)
    c2 = pltpu.async_copy(a_hbm.at[iidx], av, sem)
    c3 = pltpu.async_copy(b_hbm.at[iidx], bv, sem)
    c4 = pltpu.async_copy(c_hbm.at[iidx], cv, sem)
    c2.wait()
    pltpu.sync_copy(av, a_out.at[pl.ds(base, BPW)])
    c3.wait()
    pltpu.sync_copy(bv, b_out.at[pl.ds(base, BPW)])
    c4.wait()
    pltpu.sync_copy(cv, c_out.at[pl.ds(base, BPW)])


def _irf_body(user_hbm, theta_hbm, a_hbm, b_hbm, c_hbm, out_hbm,
              uidx, tv, av, bv, cv, ov, isem, gsem):
    wid = lax.axis_index("s") * NC + lax.axis_index("c")
    base = wid * BPW
    sl_all = pl.ds(base, BPW)
    pltpu.sync_copy(user_hbm.at[sl_all], uidx)
    cg = pltpu.async_copy(theta_hbm.at[uidx], tv, gsem)
    ca = pltpu.async_copy(a_hbm.at[sl_all], av, isem)
    cb = pltpu.async_copy(b_hbm.at[sl_all], bv, isem)
    cc = pltpu.async_copy(c_hbm.at[sl_all], cv, isem)
    ca.wait(); cb.wait(); cc.wait(); cg.wait()
    for j in range(BPW // L):
        sl = pl.ds(j * L, L)
        t = tv[sl]
        a = jnp.maximum(av[sl], 0.001)
        b = bv[sl]
        c = jnp.clip(cv[sl], 0.0, 1.0)
        sig = 1.0 / (1.0 + jnp.exp(-D_CONST * a * (t - b)))
        ov[sl] = c + (1.0 - c) * sig
    pltpu.sync_copy(ov, out_hbm.at[sl_all])


FLAT_BLOCK = 8192


def _flatten_body(x_ref, o_ref):
    o_ref[:] = x_ref[:, 0]


def _flatten_tc(x):
    n = x.shape[0]
    return pl.pallas_call(
        _flatten_body,
        grid=((n + FLAT_BLOCK - 1) // FLAT_BLOCK,),
        in_specs=[pl.BlockSpec((FLAT_BLOCK, 1), lambda i: (i, 0))],
        out_specs=pl.BlockSpec((FLAT_BLOCK,), lambda i: (i,)),
        out_shape=jax.ShapeDtypeStruct((n,), jnp.float32),
    )(x)


def kernel(user, item, theta_w, a_w, b_w, c_w):
    user = user.astype(jnp.int32)
    item = item.astype(jnp.int32)
    a_flat = a_w.reshape(-1)
    b_flat = b_w.reshape(-1)
    c_flat = c_w.reshape(-1)
    theta_flat = _flatten_tc(theta_w)
    mesh = plsc.VectorSubcoreMesh(core_axis_name="c", subcore_axis_name="s")
    vals = jax.ShapeDtypeStruct((BATCH,), jnp.float32)
    abc = pl.kernel(
        _abc_body,
        mesh=mesh,
        out_type=(vals, vals, vals),
        scratch_types=[
            pltpu.VMEM((BPW,), jnp.int32),
            pltpu.VMEM((BPW,), jnp.float32),
            pltpu.VMEM((BPW,), jnp.float32),
            pltpu.VMEM((BPW,), jnp.float32),
            pltpu.SemaphoreType.DMA,
        ],
    )
    a_v, b_v, c_v = abc(item, a_flat, b_flat, c_flat)

    irf = pl.kernel(
        _irf_body,
        mesh=mesh,
        out_type=vals,
        scratch_types=[
            pltpu.VMEM((BPW,), jnp.int32),
            pltpu.VMEM((BPW,), jnp.float32),
            pltpu.VMEM((BPW,), jnp.float32),
            pltpu.VMEM((BPW,), jnp.float32),
            pltpu.VMEM((BPW,), jnp.float32),
            pltpu.VMEM((BPW,), jnp.float32),
            pltpu.SemaphoreType.DMA,
            pltpu.SemaphoreType.DMA,
        ],
    )
    return irf(user, theta_flat, a_v, b_v, c_v)


# speedup vs baseline: 9.2053x; 9.2053x over previous
import jax
import jax.numpy as jnp
from jax import lax
from jax.experimental import pallas as pl
from jax.experimental.pallas import tpu as pltpu
from jax.experimental.pallas import tpu_sc as plsc

BATCH = 16384
NC, NS, L = 2, 16, 16
NW = NC * NS
BPW = BATCH // NW
D_CONST = 1.702


def _abc_body(item_hbm, a_hbm, b_hbm, c_hbm, a_out, b_out, c_out,
              iidx, av, bv, cv, sem):
    wid = lax.axis_index("s") * NC + lax.axis_index("c")
    base = wid * BPW
    pltpu.sync_copy(item_hbm.at[pl.ds(base, BPW)], iidx)
    c2 = pltpu.async_copy(a_hbm.at[iidx], av, sem)
    c3 = pltpu.async_copy(b_hbm.at[iidx], bv, sem)
    c4 = pltpu.async_copy(c_hbm.at[iidx], cv, sem)
    c2.wait()
    pltpu.sync_copy(av, a_out.at[pl.ds(base, BPW)])
    c3.wait()
    pltpu.sync_copy(bv, b_out.at[pl.ds(base, BPW)])
    c4.wait()
    pltpu.sync_copy(cv, c_out.at[pl.ds(base, BPW)])


def _irf_body(user_hbm, theta_hbm, a_hbm, b_hbm, c_hbm, out_hbm,
              uidx, tv, av, bv, cv, ov, isem, gsem):
    wid = lax.axis_index("s") * NC + lax.axis_index("c")
    base = wid * BPW
    sl_all = pl.ds(base, BPW)
    pltpu.sync_copy(user_hbm.at[sl_all], uidx)
    cg = pltpu.async_copy(theta_hbm.at[uidx], tv, gsem)
    ca = pltpu.async_copy(a_hbm.at[sl_all], av, isem)
    cb = pltpu.async_copy(b_hbm.at[sl_all], bv, isem)
    cc = pltpu.async_copy(c_hbm.at[sl_all], cv, isem)
    ca.wait(); cb.wait(); cc.wait(); cg.wait()
    for j in range(BPW // L):
        sl = pl.ds(j * L, L)
        t = tv[sl]
        a = jnp.maximum(av[sl], 0.001)
        b = bv[sl]
        c = jnp.clip(cv[sl], 0.0, 1.0)
        sig = 1.0 / (1.0 + jnp.exp(-D_CONST * a * (t - b)))
        ov[sl] = c + (1.0 - c) * sig
    pltpu.sync_copy(ov, out_hbm.at[sl_all])


def kernel(user, item, theta_w, a_w, b_w, c_w):
    user = user.astype(jnp.int32)
    item = item.astype(jnp.int32)
    a_flat = a_w.reshape(-1)
    b_flat = b_w.reshape(-1)
    c_flat = c_w.reshape(-1)
    theta_flat = theta_w[:, 0]
    mesh = plsc.VectorSubcoreMesh(core_axis_name="c", subcore_axis_name="s")
    vals = jax.ShapeDtypeStruct((BATCH,), jnp.float32)
    abc = pl.kernel(
        _abc_body,
        mesh=mesh,
        out_type=(vals, vals, vals),
        scratch_types=[
            pltpu.VMEM((BPW,), jnp.int32),
            pltpu.VMEM((BPW,), jnp.float32),
            pltpu.VMEM((BPW,), jnp.float32),
            pltpu.VMEM((BPW,), jnp.float32),
            pltpu.SemaphoreType.DMA,
        ],
    )
    a_v, b_v, c_v = abc(item, a_flat, b_flat, c_flat)

    irf = pl.kernel(
        _irf_body,
        mesh=mesh,
        out_type=vals,
        scratch_types=[
            pltpu.VMEM((BPW,), jnp.int32),
            pltpu.VMEM((BPW,), jnp.float32),
            pltpu.VMEM((BPW,), jnp.float32),
            pltpu.VMEM((BPW,), jnp.float32),
            pltpu.VMEM((BPW,), jnp.float32),
            pltpu.VMEM((BPW,), jnp.float32),
            pltpu.SemaphoreType.DMA,
            pltpu.SemaphoreType.DMA,
        ],
    )
    return irf(user, theta_flat, a_v, b_v, c_v)
